# baseline (speedup 1.0000x reference)
import functools

import jax
import jax.numpy as jnp
from jax import lax
from jax.experimental import pallas as pl
from jax.experimental.pallas import tpu as pltpu
from jax.experimental.pallas import tpu_sc as plsc

_TOPK_POS = 10
_TOPK_NEG = 100
_TEMP = 0.07
_NUM_CLASSES = 100

_U32 = jnp.uint32
_I32 = jnp.int32
_F32 = jnp.float32
_NEG_INF = float("-inf")
_POS_INF = float("inf")
_UNROLL = 4


def _fkey(v):
    bits = lax.bitcast_convert_type(v, _U32)
    flip = jnp.where(bits >= _U32(0x80000000), _U32(0xFFFFFFFF),
                     _U32(0x80000000))
    return bits ^ flip


def _sim_body(rows_ref, cols_ref, out_ref):
    rows = rows_ref[...]
    cols = cols_ref[...]
    row_inv = 1.0 / jnp.maximum(
        jnp.sqrt(jnp.sum(rows * rows, axis=1, keepdims=True)), 1e-12)
    col_inv = 1.0 / jnp.maximum(
        jnp.sqrt(jnp.sum(cols * cols, axis=1, keepdims=True)), 1e-12)
    cols_n = cols * col_inv
    sim = lax.dot_general(rows, cols_n, (((1,), (1,)), ((), ())),
                          preferred_element_type=_F32)
    out_ref[...] = sim * row_inv


def _compute_sim(new_feat, block_rows=512):
    b, c = new_feat.shape
    return pl.pallas_call(
        _sim_body,
        grid=(b // block_rows,),
        in_specs=[
            pl.BlockSpec((block_rows, c), lambda i: (i, 0)),
            pl.BlockSpec((b, c), lambda i: (0, 0)),
        ],
        out_specs=pl.BlockSpec((block_rows, b), lambda i: (i, 0)),
        out_shape=jax.ShapeDtypeStruct((b, b), _F32),
    )(new_feat, new_feat)


def _lane_iota():
    return lax.iota(_I32, 16)


def _splat_i(x):
    return jnp.full((16,), x, _I32)


def _scalar_from(vec, lane):
    return jnp.sum(jnp.where(_lane_iota() == lane, vec, 0))


def _make_sc_select(b, kp, kn):
    nc, ns = 2, 16
    nw = nc * ns
    rows_per = b // nw
    mesh = plsc.VectorSubcoreMesh(core_axis_name="c", subcore_axis_name="s")
    nvr = b // 16
    inv_t = _F32(1.0 / _TEMP)

    @functools.partial(
        pl.kernel, mesh=mesh,
        compiler_params=pltpu.CompilerParams(needs_layout_passes=False),
        out_type=jax.ShapeDtypeStruct((b * 16,), _F32),
        scratch_types=[
            pltpu.VMEM((b,), _I32),
            pltpu.VMEM((b,), _F32),
            pltpu.VMEM((b,), _F32),
            pltpu.VMEM((256,), _I32),
            pltpu.VMEM((256,), _I32),
            pltpu.VMEM((b + 16,), _U32),
            pltpu.VMEM((b + 16,), _F32),
            pltpu.VMEM((b + 16,), _F32),
            pltpu.VMEM((b + 16,), _F32),
            pltpu.VMEM((rows_per * 16,), _F32),
            pltpu.SemaphoreType.DMA,
            pltpu.SemaphoreType.DMA,
        ],
    )
    def sc_select(sim_hbm, tgt_hbm, out_hbm, tcol_v, row_a, row_b,
                  hist1, hist2, b1k, b1v, b2v, posv, outv, sema, semb):
        wid = lax.axis_index("s") * nc + lax.axis_index("c")
        base = wid * rows_per
        pltpu.sync_copy(tgt_hbm, tcol_v)
        iota = _lane_iota()

        def fetch(ridx, buf, sem):
            rclamp = jnp.minimum(ridx, rows_per - 1)
            pltpu.async_copy(
                sim_hbm.at[pl.ds((base + rclamp) * b, b)], buf, sem)

        def dwait(ridx, buf, sem):
            rclamp = jnp.minimum(ridx, rows_per - 1)
            pltpu.make_async_copy(
                sim_hbm.at[pl.ds((base + rclamp) * b, b)], buf, sem).wait()

        def process(r, row_ref):
            trow = plsc.load_gather(tcol_v, [_splat_i(0) + (base + r)])
            for h in range(16):
                hist1[pl.ds(h * 16, 16)] = jnp.zeros((16,), _I32)
                hist2[pl.ds(h * 16, 16)] = jnp.zeros((16,), _I32)
            ones = jnp.ones((16,), _I32)

            def scan_a(jj, carry):
                npos, = carry
                for u in range(_UNROLL):
                    j = jj * _UNROLL + u
                    v = row_ref[pl.ds(j * 16, 16)]
                    tc = tcol_v[pl.ds(j * 16, 16)]
                    mpos = tc == trow
                    key = _fkey(v)
                    bkt = (key >> _U32(24)).astype(_I32)
                    plsc.addupdate_scatter(hist1, [bkt], ones,
                                           mask=jnp.logical_not(mpos))
                    plsc.store_compressed(posv.at[pl.ds(npos, 16)], v,
                                          mask=mpos)
                    npos = npos + jnp.max(
                        plsc.all_reduce_population_count(mpos))
                return (npos,)

            (npos,) = lax.fori_loop(0, nvr // _UNROLL, scan_a, (jnp.int32(0),))

            b1s, above1 = _find2(hist1, jnp.int32(kn))

            def scan_b(jj, carry):
                nb1, s1 = carry
                for u in range(_UNROLL):
                    j = jj * _UNROLL + u
                    v = row_ref[pl.ds(j * 16, 16)]
                    tc = tcol_v[pl.ds(j * 16, 16)]
                    mneg = tc != trow
                    key = _fkey(v)
                    bkt = (key >> _U32(24)).astype(_I32)
                    m_above = mneg & (bkt > b1s)
                    s1 = s1 + jnp.where(m_above, jnp.exp(v * inv_t), 0.0)
                    m_band = mneg & (bkt == b1s)
                    plsc.store_compressed(b1k.at[pl.ds(nb1, 16)], key,
                                          mask=m_band)
                    plsc.store_compressed(b1v.at[pl.ds(nb1, 16)], v,
                                          mask=m_band)
                    nb1 = nb1 + jnp.max(
                        plsc.all_reduce_population_count(m_band))
                return nb1, s1

            nb1, s1vec = lax.fori_loop(
                0, nvr // _UNROLL, scan_b,
                (jnp.int32(0), jnp.zeros((16,), _F32)))

            nb1v = (nb1 + 15) // 16

            def scan_l2(j, carry):
                del carry
                bk = b1k[pl.ds(j * 16, 16)]
                valid = (j * 16 + iota) < nb1
                b2 = ((bk >> _U32(16)) & _U32(0xFF)).astype(_I32)
                plsc.addupdate_scatter(hist2, [b2], ones, mask=valid)
                return (jnp.int32(0),)

            lax.fori_loop(0, nb1v, scan_l2, (jnp.int32(0),))
            b2s, above2 = _find2(hist2, kn - above1)

            def scan_c(j, carry):
                nb2, s2 = carry
                bk = b1k[pl.ds(j * 16, 16)]
                bv = b1v[pl.ds(j * 16, 16)]
                valid = (j * 16 + iota) < nb1
                b2 = ((bk >> _U32(16)) & _U32(0xFF)).astype(_I32)
                m_above = valid & (b2 > b2s)
                s2 = s2 + jnp.where(m_above, jnp.exp(bv * inv_t), 0.0)
                m_band = valid & (b2 == b2s)
                plsc.store_compressed(b2v.at[pl.ds(nb2, 16)], bv, mask=m_band)
                cnt = jnp.max(plsc.all_reduce_population_count(m_band))
                return nb2 + cnt, s2

            nb2, s2vec = lax.fori_loop(
                0, nb1v, scan_c, (jnp.int32(0), jnp.zeros((16,), _F32)))

            nb2v = (nb2 + 15) // 16
            r_need = (kn - above1 - above2).astype(_F32)

            def ext_cond(carry):
                r, _ = carry
                return r > 0.5

            def ext_body(carry):
                r, sb = carry

                def mx(j, m):
                    bv = b2v[pl.ds(j * 16, 16)]
                    valid = (j * 16 + iota) < nb2
                    return jnp.maximum(
                        m, jnp.max(jnp.where(valid, bv, _NEG_INF)))

                m = lax.fori_loop(0, nb2v, mx, _F32(_NEG_INF))

                def cnt_rm(j, c):
                    bv = b2v[pl.ds(j * 16, 16)]
                    valid = (j * 16 + iota) < nb2
                    eq = valid & (bv == m)
                    b2v[pl.ds(j * 16, 16)] = jnp.where(eq, _NEG_INF, bv)
                    return c + jnp.max(plsc.all_reduce_population_count(eq))

                ceq = lax.fori_loop(0, nb2v, cnt_rm, jnp.int32(0))
                ceq = jnp.maximum(ceq.astype(_F32), 1.0)
                take = jnp.minimum(r, ceq)
                mexp = jnp.sum(
                    jnp.where(iota == 0, jnp.exp(jnp.full((16,), m) * inv_t),
                              0.0))
                return r - take, sb + take * mexp

            _, s_band = lax.while_loop(ext_cond, ext_body,
                                       (r_need, _F32(0.0)))
            s_total = jnp.sum(s1vec) + jnp.sum(s2vec) + s_band

            npv = (npos + 15) // 16

            def pmerge(j, w):
                v = posv[pl.ds(j * 16, 16)]
                valid = (j * 16 + iota) < npos
                v = jnp.where(valid, v, _POS_INF)
                vs, _ = plsc.sort_key_val(v, v)
                lo = jnp.minimum(w, lax.rev(vs, (0,)))
                wlo, _ = plsc.sort_key_val(lo, lo)
                return wlo

            w = lax.fori_loop(0, npv, pmerge,
                              jnp.full((16,), _POS_INF, _F32))

            out_vec = jnp.where(iota == 15, jnp.full((16,), s_total), w)
            outv[pl.ds(r * 16, 16)] = out_vec

        fetch(0, row_a, sema)

        def row_pair(i, carry):
            del carry
            r = i * 2
            dwait(r, row_a, sema)
            fetch(r + 1, row_b, semb)
            process(r, row_a)
            dwait(r + 1, row_b, semb)
            fetch(r + 2, row_a, sema)
            process(r + 1, row_b)
            return (jnp.int32(0),)

        lax.fori_loop(0, rows_per // 2, row_pair, (jnp.int32(0),))
        dwait(rows_per - 1, row_a, sema)
        pltpu.sync_copy(outv, out_hbm.at[pl.ds(base * 16, rows_per * 16)])

    return sc_select


def _find2(hist, krem):
    iota = _lane_iota()
    run = jnp.int32(0)
    bstar = jnp.int32(-1)
    above = jnp.int32(0)
    for h in range(15, -1, -1):
        hv = hist[pl.ds(h * 16, 16)]
        rc = lax.rev(hv, (0,))
        cs = jnp.cumsum(rc)
        crossed = (run + cs) >= krem
        hit = jnp.any(crossed) & (bstar < 0)
        lane = jnp.max(plsc.all_reduce_ffs(crossed))
        bkt_h = h * 16 + 15 - lane
        cs_at = jnp.sum(jnp.where(iota == lane, cs, 0))
        h_at = jnp.sum(jnp.where(iota == lane, rc, 0))
        bstar = jnp.where(hit, bkt_h, bstar)
        above = jnp.where(hit, run + cs_at - h_at, above)
        run = run + jnp.sum(jnp.where(iota == 15, cs, 0))
    return bstar, above


def _loss_body(pack_ref, out_ref, *, kp):
    pack = pack_ref[...]
    s = pack[:, 15:16]
    inv_t = _F32(1.0 / _TEMP)
    lsum = jnp.zeros_like(s)
    lcnt = jnp.zeros_like(s)
    for j in range(kp):
        p = pack[:, j:j + 1]
        fm = jnp.log(1.0 + s * jnp.exp(-p * inv_t))
        lsum += fm
        lcnt += (fm != 0.0).astype(_F32)
    out_ref[...] = (jnp.sum(lsum) /
                    jnp.maximum(jnp.sum(lcnt), 1.0)).reshape(1, 1)


def _loss(pack, kp):
    b = pack.shape[0]
    return pl.pallas_call(
        functools.partial(_loss_body, kp=kp),
        in_specs=[pl.BlockSpec((b, 16), lambda: (0, 0))],
        out_specs=pl.BlockSpec((1, 1), lambda: (0, 0)),
        out_shape=jax.ShapeDtypeStruct((1, 1), _F32),
    )(pack)


def kernel(old_feat, new_feat, target):
    del old_feat
    b, _ = new_feat.shape
    kp = min(_TOPK_POS, -(-b // _NUM_CLASSES) - 1, b - 1) if _TOPK_POS > 0 else 1
    kn = min(_TOPK_NEG, b - 1) if _TOPK_NEG > 0 else 1
    sim = _compute_sim(new_feat)
    sel = _make_sc_select(b, kp, kn)
    pack = sel(sim.reshape(-1), target.astype(_I32)).reshape(b, 16)
    return _loss(pack, kp).reshape(())


# speedup vs baseline: 2.9127x; 2.9127x over previous
import functools

import jax
import jax.numpy as jnp
from jax.experimental import pallas as pl
from jax.experimental.pallas import tpu as pltpu

_TOPK_POS = 10
_TOPK_NEG = 100
_TEMP = 0.07
_NUM_CLASSES = 100

_U32 = jnp.uint32
_KEY_NEG_INF = 0x007FFFFF


def _float_key(bits):
    flip = jnp.where(bits >= _U32(0x80000000), _U32(0xFFFFFFFF), _U32(0x80000000))
    return bits ^ flip


def _key_to_float(key):
    bits = jnp.where(key >= _U32(0x80000000), key ^ _U32(0x80000000), ~key)
    return jax.lax.bitcast_convert_type(bits, jnp.float32)


def _kth_largest(keys, k):
    rows = keys.shape[0]
    t = jnp.zeros((rows, 1), _U32)
    kf = jnp.float32(k)
    for b in range(31, -1, -1):
        cand = t | _U32(1 << b)
        cnt = jnp.sum((keys >= cand).astype(jnp.float32), axis=1, keepdims=True)
        t = jnp.where(cnt >= kf, cand, t)
    return t


def _body(rows_ref, cols_ref, trow_ref, tcol_ref, out_ref, acc_sum, acc_cnt,
          *, nblocks, kp, kn):
    i = pl.program_id(0)
    rows = rows_ref[...]
    cols = cols_ref[...]
    trow = trow_ref[...]
    tcol = tcol_ref[...]

    row_inv = 1.0 / jnp.maximum(
        jnp.sqrt(jnp.sum(rows * rows, axis=1, keepdims=True)), 1e-12)
    col_inv = 1.0 / jnp.maximum(
        jnp.sqrt(jnp.sum(cols * cols, axis=1, keepdims=True)), 1e-12)
    cols_n = cols * col_inv
    sim = jax.lax.dot_general(
        rows, cols_n, (((1,), (1,)), ((), ())),
        preferred_element_type=jnp.float32)
    sim = sim * row_inv

    pos = trow == tcol

    bits = jax.lax.bitcast_convert_type(sim, _U32)
    key = _float_key(bits)
    keys_neg = jnp.where(pos, _U32(_KEY_NEG_INF), key)

    tn = _kth_largest(keys_neg, kn)

    inv_t = jnp.float32(1.0 / _TEMP)
    tn_val = _key_to_float(tn)

    exp_n = jnp.exp(sim * inv_t)
    gt_n = keys_neg > tn
    cnt_gt = jnp.sum(gt_n.astype(jnp.float32), axis=1, keepdims=True)
    s_neg = (jnp.sum(jnp.where(gt_n, exp_n, 0.0), axis=1, keepdims=True)
             + (jnp.float32(kn) - cnt_gt) * jnp.exp(tn_val * inv_t))

    masked = jnp.where(pos, sim, jnp.float32(jnp.inf))
    remaining = jnp.full((sim.shape[0], 1), jnp.float32(kp))
    lsum = jnp.zeros_like(remaining)
    lcnt = jnp.zeros_like(remaining)
    for _ in range(kp):
        m = jnp.min(masked, axis=1, keepdims=True)
        eq = masked == m
        ceq = jnp.sum(eq.astype(jnp.float32), axis=1, keepdims=True)
        take = jnp.minimum(remaining, ceq)
        fm = jnp.log(1.0 + s_neg * jnp.exp(-m * inv_t))
        lsum += take * fm
        lcnt += take * (fm != 0.0).astype(jnp.float32)
        masked = jnp.where(eq, jnp.float32(jnp.inf), masked)
        remaining -= take

    block_sum = jnp.sum(lsum).reshape(1, 1)
    block_cnt = jnp.sum(lcnt).reshape(1, 1)

    @pl.when(i == 0)
    def _():
        acc_sum[...] = jnp.zeros_like(acc_sum)
        acc_cnt[...] = jnp.zeros_like(acc_cnt)

    acc_sum[...] += block_sum
    acc_cnt[...] += block_cnt

    @pl.when(i == nblocks - 1)
    def _():
        out_ref[...] = acc_sum[...] / jnp.maximum(acc_cnt[...], 1.0)


def _run(new_feat, target, *, block_rows=256, interpret=False):
    b, c = new_feat.shape
    kp = min(_TOPK_POS, -(-b // _NUM_CLASSES) - 1, b - 1) if _TOPK_POS > 0 else 1
    kn = min(_TOPK_NEG, b - 1) if _TOPK_NEG > 0 else 1
    tgt = target.astype(jnp.float32)
    nblocks = b // block_rows
    out = pl.pallas_call(
        functools.partial(_body, nblocks=nblocks, kp=kp, kn=kn),
        grid=(nblocks,),
        in_specs=[
            pl.BlockSpec((block_rows, c), lambda i: (i, 0)),
            pl.BlockSpec((b, c), lambda i: (0, 0)),
            pl.BlockSpec((block_rows, 1), lambda i: (i, 0)),
            pl.BlockSpec((1, b), lambda i: (0, 0)),
        ],
        out_specs=pl.BlockSpec((1, 1), lambda i: (0, 0)),
        out_shape=jax.ShapeDtypeStruct((1, 1), jnp.float32),
        scratch_shapes=[pltpu.VMEM((1, 1), jnp.float32),
                        pltpu.VMEM((1, 1), jnp.float32)],
        interpret=interpret,
    )(new_feat, new_feat, tgt.reshape(b, 1), tgt.reshape(1, b))
    return out.reshape(())


def kernel(old_feat, new_feat, target):
    del old_feat
    return _run(new_feat, target, block_rows=256)
